# Initial kernel scaffold; baseline (speedup 1.0000x reference)
import jax
import jax.numpy as jnp
from jax.experimental import pallas as pl


def kernel(x, Wr, W1, W2):
    raise NotImplementedError("write your pallas kernel here")


# speedup vs baseline: 1.2988x; 1.2988x over previous
import functools

import jax
import jax.numpy as jnp
from jax import lax
from jax.experimental import pallas as pl
from jax.experimental.pallas import tpu as pltpu
from jax.experimental.pallas import tpu_sc as plsc

E = 8
K = 2
D = 1024
F = 4096
T = 8192
C = 2560
S = E * C
TK = T * K

TT = 1024
TC_C = 512

NC, NS, L = 2, 16, 16
NW = NC * NS

_SC_MESH = dict(mesh=plsc.VectorSubcoreMesh(core_axis_name="c",
                                            subcore_axis_name="s",
                                            num_cores=NC,
                                            num_subcores=NS))


def _router_body(x_ref, wr_ref, s0r_ref, s1r_ref, s0c_ref, s1c_ref,
                 w0_ref, w1_ref, carry_ref):
    i = pl.program_id(0)

    @pl.when(i == 0)
    def _():
        carry_ref[...] = jnp.zeros_like(carry_ref)

    xt = x_ref[...]
    wr = wr_ref[...]
    logits = jnp.dot(xt, wr, preferred_element_type=jnp.float32)
    lane = lax.broadcasted_iota(jnp.int32, logits.shape, 1)
    valid = lane < E
    logits = jnp.where(valid, logits, jnp.float32(-1e30))
    mx = jnp.max(logits, axis=1, keepdims=True)
    ex = jnp.where(valid, jnp.exp(logits - mx), 0.0)
    probs = ex / jnp.sum(ex, axis=1, keepdims=True)

    p0 = jnp.max(probs, axis=1, keepdims=True)
    e0 = jnp.min(jnp.where((probs == p0) & valid, lane, 128),
                 axis=1, keepdims=True)
    m0 = lane == e0
    probs2 = jnp.where(m0, jnp.float32(-1.0), probs)
    p1 = jnp.max(probs2, axis=1, keepdims=True)
    e1 = jnp.min(jnp.where((probs2 == p1) & valid, lane, 128),
                 axis=1, keepdims=True)
    m1 = lane == e1
    denom = p0 + p1 + jnp.float32(1e-9)
    g0 = p0 / denom
    g1 = p1 / denom

    two_hot = m0.astype(jnp.float32) + m1.astype(jnp.float32)
    row = lax.broadcasted_iota(jnp.int32, (TT, TT), 0)
    col = lax.broadcasted_iota(jnp.int32, (TT, TT), 1)
    tril = (col < row).astype(jnp.float32)
    cex = jnp.dot(tril, two_hot, preferred_element_type=jnp.float32)
    cex = cex + carry_ref[...]
    carry_ref[...] = carry_ref[...] + jnp.sum(two_hot, axis=0, keepdims=True)

    pos0 = jnp.sum(jnp.where(m0, cex, 0.0), axis=1, keepdims=True)
    pos1 = jnp.sum(jnp.where(m1, cex, 0.0), axis=1, keepdims=True)
    keep0 = pos0 < jnp.float32(C)
    keep1 = pos1 < jnp.float32(C)
    slot0 = e0 * C + pos0.astype(jnp.int32)
    slot1 = e1 * C + pos1.astype(jnp.int32)
    s0r_ref[...] = jnp.where(keep0, slot0, S)
    s1r_ref[...] = jnp.where(keep1, slot1, S)
    s0c_ref[...] = jnp.where(keep0, slot0, 0)
    s1c_ref[...] = jnp.where(keep1, slot1, 0)
    w0_ref[...] = jnp.where(keep0, g0, 0.0)
    w1_ref[...] = jnp.where(keep1, g1, 0.0)


def _router(x, wr_pad):
    i32 = jax.ShapeDtypeStruct((T, 1), jnp.int32)
    f32 = jax.ShapeDtypeStruct((T, 1), jnp.float32)
    return pl.pallas_call(
        _router_body,
        grid=(T // TT,),
        in_specs=[pl.BlockSpec((TT, D), lambda i: (i, 0)),
                  pl.BlockSpec((D, 128), lambda i: (0, 0))],
        out_specs=[pl.BlockSpec((TT, 1), lambda i: (i, 0))] * 6,
        out_shape=[i32, i32, i32, i32, f32, f32],
        scratch_shapes=[pltpu.VMEM((1, 128), jnp.float32)],
    )(x, wr_pad)


def _invert_body(s0_hbm, s1_hbm, tfs_hbm, table_v, slots_v):
    cid = lax.axis_index("c")
    sid = lax.axis_index("s")

    @pl.when((cid == 0) & (sid == 0))
    def _():
        pltpu.sync_copy(s0_hbm, slots_v.at[pl.ds(0, T)])
        pltpu.sync_copy(s1_hbm, slots_v.at[pl.ds(T, T)])

        def zero(i, carry):
            table_v[pl.ds(i * L, L)] = jnp.zeros((L,), jnp.int32)
            return carry
        lax.fori_loop(0, S // L, zero, 0)

        def scat(i, carry):
            tok = lax.iota(jnp.int32, L) + (i % (T // L)) * L
            idx = slots_v[pl.ds(i * L, L)]
            plsc.store_scatter(table_v, [idx], tok, mask=idx < S)
            return carry
        lax.fori_loop(0, TK // L, scat, 0)
        pltpu.sync_copy(table_v, tfs_hbm)


def _invert(s0, s1):
    return pl.kernel(
        _invert_body,
        out_type=jax.ShapeDtypeStruct((S,), jnp.int32),
        scratch_types=[pltpu.VMEM((S,), jnp.int32),
                       pltpu.VMEM((TK,), jnp.int32)],
        compiler_params=pltpu.CompilerParams(needs_layout_passes=False),
        **_SC_MESH,
    )(s0, s1)


_ROWS_W = S // NW
_CH = 64


def _dispatch_body(x_hbm, tfs_hbm, buf_hbm, idx_v, rows_v, sem):
    cid = lax.axis_index("c")
    sid = lax.axis_index("s")
    wid = sid * NC + cid
    base = wid * _ROWS_W
    pltpu.sync_copy(tfs_hbm.at[pl.ds(base, _ROWS_W)], idx_v)

    def chunk(ci, carry):
        pltpu.async_copy(x_hbm.at[idx_v.at[pl.ds(ci * _CH, _CH)]],
                         rows_v, sem).wait()
        pltpu.sync_copy(rows_v, buf_hbm.at[pl.ds(base + ci * _CH, _CH)])
        return carry
    lax.fori_loop(0, _ROWS_W // _CH, chunk, 0)


def _dispatch(x, tfs):
    return pl.kernel(
        _dispatch_body,
        out_type=jax.ShapeDtypeStruct((S, D), jnp.float32),
        scratch_types=[pltpu.VMEM((_ROWS_W,), jnp.int32),
                       pltpu.VMEM((_CH, D), jnp.float32),
                       pltpu.SemaphoreType.DMA],
        **_SC_MESH,
    )(x, tfs)


def _ffn_body(buf_ref, w1_ref, w2_ref, out_ref):
    a = buf_ref[0].astype(jnp.bfloat16)
    h = jnp.maximum(
        jnp.dot(a, w1_ref[0], preferred_element_type=jnp.float32),
        0.0).astype(jnp.bfloat16)
    out_ref[0] = jnp.dot(h, w2_ref[0], preferred_element_type=jnp.float32)


def _ffn(buf3, w1b, w2b):
    return pl.pallas_call(
        _ffn_body,
        grid=(E, C // TC_C),
        in_specs=[pl.BlockSpec((1, TC_C, D), lambda e, c: (e, c, 0)),
                  pl.BlockSpec((1, D, F), lambda e, c: (e, 0, 0)),
                  pl.BlockSpec((1, F, D), lambda e, c: (e, 0, 0))],
        out_specs=pl.BlockSpec((1, TC_C, D), lambda e, c: (e, c, 0)),
        out_shape=jax.ShapeDtypeStruct((E, C, D), jnp.float32),
    )(buf3, w1b, w2b)


_TOK_W = T // NW
_CHT = 16

_GDN = lax.GatherDimensionNumbers(offset_dims=(), collapsed_slice_dims=(0,),
                                  start_index_map=(0,))


def _lane_broadcast(vec, t):
    idx = jnp.full((L, 1), t, dtype=jnp.int32)
    return lax.gather(vec, idx, _GDN, (1,),
                      mode=lax.GatherScatterMode.PROMISE_IN_BOUNDS)


def _combine_body(eo_hbm, s0_hbm, s1_hbm, w0_hbm, w1_hbm, y_hbm,
                  i0_v, i1_v, w0_v, w1_v, a_v, b_v, y_v, sem_a, sem_b):
    cid = lax.axis_index("c")
    sid = lax.axis_index("s")
    wid = sid * NC + cid
    base = wid * _TOK_W
    pltpu.sync_copy(s0_hbm.at[pl.ds(base, _TOK_W)], i0_v)
    pltpu.sync_copy(s1_hbm.at[pl.ds(base, _TOK_W)], i1_v)
    pltpu.sync_copy(w0_hbm.at[pl.ds(base, _TOK_W)], w0_v)
    pltpu.sync_copy(w1_hbm.at[pl.ds(base, _TOK_W)], w1_v)

    def chunk(ci, carry):
        ca = pltpu.async_copy(eo_hbm.at[i0_v.at[pl.ds(ci * _CHT, _CHT)]],
                              a_v, sem_a)
        cb = pltpu.async_copy(eo_hbm.at[i1_v.at[pl.ds(ci * _CHT, _CHT)]],
                              b_v, sem_b)
        ca.wait()
        cb.wait()
        w0c = w0_v[pl.ds(ci * _CHT, L)]
        w1c = w1_v[pl.ds(ci * _CHT, L)]
        for t in range(_CHT):
            w0s = _lane_broadcast(w0c, t)
            w1s = _lane_broadcast(w1c, t)

            def feat(j, carry2, _t=t, _w0=w0s, _w1=w1s):
                a = a_v[_t, pl.ds(j * L, L)]
                b = b_v[_t, pl.ds(j * L, L)]
                y_v[_t, pl.ds(j * L, L)] = _w0 * a + _w1 * b
                return carry2
            lax.fori_loop(0, D // L, feat, 0)
        pltpu.sync_copy(y_v, y_hbm.at[pl.ds(base + ci * _CHT, _CHT)])
        return carry
    lax.fori_loop(0, _TOK_W // _CHT, chunk, 0)


def _combine(eo, s0c, s1c, w0, w1):
    return pl.kernel(
        _combine_body,
        out_type=jax.ShapeDtypeStruct((T, D), jnp.float32),
        scratch_types=[pltpu.VMEM((_TOK_W,), jnp.int32),
                       pltpu.VMEM((_TOK_W,), jnp.int32),
                       pltpu.VMEM((_TOK_W,), jnp.float32),
                       pltpu.VMEM((_TOK_W,), jnp.float32),
                       pltpu.VMEM((_CHT, D), jnp.float32),
                       pltpu.VMEM((_CHT, D), jnp.float32),
                       pltpu.VMEM((_CHT, D), jnp.float32),
                       pltpu.SemaphoreType.DMA,
                       pltpu.SemaphoreType.DMA],
        **_SC_MESH,
    )(eo, s0c, s1c, w0, w1)


def kernel(x, Wr, W1, W2):
    wr_pad = jnp.pad(Wr, ((0, 0), (0, 128 - E)))
    s0r, s1r, s0c, s1c, w0, w1 = _router(x, wr_pad)
    tfs = _invert(s0r.reshape(T), s1r.reshape(T))
    buf = _dispatch(x, tfs)
    eo = _ffn(buf.reshape(E, C, D), W1.astype(jnp.bfloat16),
              W2.astype(jnp.bfloat16))
    return _combine(eo.reshape(S, D), s0c.reshape(T), s1c.reshape(T),
                    w0.reshape(T), w1.reshape(T))


# speedup vs baseline: 1.3323x; 1.0258x over previous
import functools

import jax
import jax.numpy as jnp
from jax import lax
from jax.experimental import pallas as pl
from jax.experimental.pallas import tpu as pltpu
from jax.experimental.pallas import tpu_sc as plsc

E = 8
K = 2
D = 1024
F = 4096
T = 8192
C = 2560
S = E * C
TK = T * K

TT = 1024
TC_C = 512

NC, NS, L = 2, 16, 16
NW = NC * NS

_SC_MESH = dict(mesh=plsc.VectorSubcoreMesh(core_axis_name="c",
                                            subcore_axis_name="s",
                                            num_cores=NC,
                                            num_subcores=NS))


def _router_body(x_ref, wr_ref, s0r_ref, s1r_ref, s0c_ref, s1c_ref,
                 w0_ref, w1_ref, carry_ref):
    i = pl.program_id(0)

    @pl.when(i == 0)
    def _():
        carry_ref[...] = jnp.zeros_like(carry_ref)

    xt = x_ref[...]
    wr = wr_ref[...]
    logits = jnp.dot(xt, wr, preferred_element_type=jnp.float32)
    lane = lax.broadcasted_iota(jnp.int32, logits.shape, 1)
    valid = lane < E
    logits = jnp.where(valid, logits, jnp.float32(-1e30))
    mx = jnp.max(logits, axis=1, keepdims=True)
    ex = jnp.where(valid, jnp.exp(logits - mx), 0.0)
    probs = ex / jnp.sum(ex, axis=1, keepdims=True)

    p0 = jnp.max(probs, axis=1, keepdims=True)
    e0 = jnp.min(jnp.where((probs == p0) & valid, lane, 128),
                 axis=1, keepdims=True)
    m0 = lane == e0
    probs2 = jnp.where(m0, jnp.float32(-1.0), probs)
    p1 = jnp.max(probs2, axis=1, keepdims=True)
    e1 = jnp.min(jnp.where((probs2 == p1) & valid, lane, 128),
                 axis=1, keepdims=True)
    m1 = lane == e1
    denom = p0 + p1 + jnp.float32(1e-9)
    g0 = p0 / denom
    g1 = p1 / denom

    two_hot = m0.astype(jnp.float32) + m1.astype(jnp.float32)
    row = lax.broadcasted_iota(jnp.int32, (TT, TT), 0)
    col = lax.broadcasted_iota(jnp.int32, (TT, TT), 1)
    tril = (col < row).astype(jnp.float32)
    cex = jnp.dot(tril, two_hot, preferred_element_type=jnp.float32)
    cex = cex + carry_ref[...]
    carry_ref[...] = carry_ref[...] + jnp.sum(two_hot, axis=0, keepdims=True)

    pos0 = jnp.sum(jnp.where(m0, cex, 0.0), axis=1, keepdims=True)
    pos1 = jnp.sum(jnp.where(m1, cex, 0.0), axis=1, keepdims=True)
    keep0 = pos0 < jnp.float32(C)
    keep1 = pos1 < jnp.float32(C)
    slot0 = e0 * C + pos0.astype(jnp.int32)
    slot1 = e1 * C + pos1.astype(jnp.int32)
    s0r_ref[...] = jnp.where(keep0, slot0, S)
    s1r_ref[...] = jnp.where(keep1, slot1, S)
    s0c_ref[...] = jnp.where(keep0, slot0, 0)
    s1c_ref[...] = jnp.where(keep1, slot1, 0)
    w0_ref[...] = jnp.where(keep0, g0, 0.0)
    w1_ref[...] = jnp.where(keep1, g1, 0.0)


def _router(x, wr_pad):
    i32 = jax.ShapeDtypeStruct((T, 1), jnp.int32)
    f32 = jax.ShapeDtypeStruct((T, 1), jnp.float32)
    return pl.pallas_call(
        _router_body,
        grid=(T // TT,),
        in_specs=[pl.BlockSpec((TT, D), lambda i: (i, 0)),
                  pl.BlockSpec((D, 128), lambda i: (0, 0))],
        out_specs=[pl.BlockSpec((TT, 1), lambda i: (i, 0))] * 6,
        out_shape=[i32, i32, i32, i32, f32, f32],
        scratch_shapes=[pltpu.VMEM((1, 128), jnp.float32)],
    )(x, wr_pad)


def _invert_body(s0_hbm, s1_hbm, tfs_hbm, table_v, slots_v):
    cid = lax.axis_index("c")
    sid = lax.axis_index("s")

    @pl.when((cid == 0) & (sid == 0))
    def _():
        pltpu.sync_copy(s0_hbm, slots_v.at[pl.ds(0, T)])
        pltpu.sync_copy(s1_hbm, slots_v.at[pl.ds(T, T)])

        def zero(i, carry):
            table_v[pl.ds(i * L, L)] = jnp.zeros((L,), jnp.int32)
            return carry
        lax.fori_loop(0, S // L, zero, 0)

        def scat(i, carry):
            tok = lax.iota(jnp.int32, L) + (i % (T // L)) * L
            idx = slots_v[pl.ds(i * L, L)]
            plsc.store_scatter(table_v, [idx], tok, mask=idx < S)
            return carry
        lax.fori_loop(0, TK // L, scat, 0)
        pltpu.sync_copy(table_v, tfs_hbm)


def _invert(s0, s1):
    return pl.kernel(
        _invert_body,
        out_type=jax.ShapeDtypeStruct((S,), jnp.int32),
        scratch_types=[pltpu.VMEM((S,), jnp.int32),
                       pltpu.VMEM((TK,), jnp.int32)],
        compiler_params=pltpu.CompilerParams(needs_layout_passes=False),
        **_SC_MESH,
    )(s0, s1)


_ROWS_W = S // NW
_CH = 40
_NCH = _ROWS_W // _CH


def _dispatch_body(x_hbm, tfs_hbm, buf_hbm, idx_v, rows0, rows1, sem0, sem1):
    cid = lax.axis_index("c")
    sid = lax.axis_index("s")
    wid = sid * NC + cid
    base = wid * _ROWS_W
    pltpu.sync_copy(tfs_hbm.at[pl.ds(base, _ROWS_W)], idx_v)

    def gather(ci, rows, sem):
        pltpu.async_copy(x_hbm.at[idx_v.at[pl.ds(ci * _CH, _CH)]], rows, sem)

    def drain(ci, rows, sem):
        pltpu.make_async_copy(x_hbm.at[idx_v.at[pl.ds(ci * _CH, _CH)]],
                              rows, sem).wait()

    gather(0, rows0, sem0)

    @pl.loop(0, _NCH, step=2)
    def _(ci):
        gather(ci + 1, rows1, sem1)
        drain(ci, rows0, sem0)
        pltpu.sync_copy(rows0, buf_hbm.at[pl.ds(base + ci * _CH, _CH)])

        @pl.when(ci + 2 < _NCH)
        def _():
            gather(ci + 2, rows0, sem0)
        drain(ci + 1, rows1, sem1)
        pltpu.sync_copy(rows1, buf_hbm.at[pl.ds(base + (ci + 1) * _CH, _CH)])


def _dispatch(x, tfs):
    return pl.kernel(
        _dispatch_body,
        out_type=jax.ShapeDtypeStruct((S, D), jnp.float32),
        scratch_types=[pltpu.VMEM((_ROWS_W,), jnp.int32),
                       pltpu.VMEM((_CH, D), jnp.float32),
                       pltpu.VMEM((_CH, D), jnp.float32),
                       pltpu.SemaphoreType.DMA,
                       pltpu.SemaphoreType.DMA],
        **_SC_MESH,
    )(x, tfs)


def _ffn_body(buf_ref, w1_ref, w2_ref, out_ref):
    a = buf_ref[0].astype(jnp.bfloat16)
    h = jnp.maximum(
        jnp.dot(a, w1_ref[0], preferred_element_type=jnp.float32),
        0.0).astype(jnp.bfloat16)
    out_ref[0] = jnp.dot(h, w2_ref[0], preferred_element_type=jnp.float32)


def _ffn(buf3, w1b, w2b):
    return pl.pallas_call(
        _ffn_body,
        grid=(E, C // TC_C),
        in_specs=[pl.BlockSpec((1, TC_C, D), lambda e, c: (e, c, 0)),
                  pl.BlockSpec((1, D, F), lambda e, c: (e, 0, 0)),
                  pl.BlockSpec((1, F, D), lambda e, c: (e, 0, 0))],
        out_specs=pl.BlockSpec((1, TC_C, D), lambda e, c: (e, c, 0)),
        out_shape=jax.ShapeDtypeStruct((E, C, D), jnp.float32),
    )(buf3, w1b, w2b)


_TOK_W = T // NW
_CHT = 16

_GDN = lax.GatherDimensionNumbers(offset_dims=(), collapsed_slice_dims=(0,),
                                  start_index_map=(0,))


def _lane_broadcast(vec, t):
    idx = jnp.full((L, 1), t, dtype=jnp.int32)
    return lax.gather(vec, idx, _GDN, (1,),
                      mode=lax.GatherScatterMode.PROMISE_IN_BOUNDS)


_NCHT = _TOK_W // _CHT


def _combine_body(eo_hbm, s0_hbm, s1_hbm, w0_hbm, w1_hbm, y_hbm,
                  i0_v, i1_v, w0_v, w1_v,
                  a0_v, b0_v, y0_v, a1_v, b1_v, y1_v,
                  sem0, sem1):
    cid = lax.axis_index("c")
    sid = lax.axis_index("s")
    wid = sid * NC + cid
    base = wid * _TOK_W
    pltpu.sync_copy(s0_hbm.at[pl.ds(base, _TOK_W)], i0_v)
    pltpu.sync_copy(s1_hbm.at[pl.ds(base, _TOK_W)], i1_v)
    pltpu.sync_copy(w0_hbm.at[pl.ds(base, _TOK_W)], w0_v)
    pltpu.sync_copy(w1_hbm.at[pl.ds(base, _TOK_W)], w1_v)

    def gather(ci, a_v, b_v, sem):
        pltpu.async_copy(eo_hbm.at[i0_v.at[pl.ds(ci * _CHT, _CHT)]], a_v, sem)
        pltpu.async_copy(eo_hbm.at[i1_v.at[pl.ds(ci * _CHT, _CHT)]], b_v, sem)

    def drain(ci, a_v, b_v, sem):
        pltpu.make_async_copy(
            eo_hbm.at[i0_v.at[pl.ds(ci * _CHT, _CHT)]], a_v, sem).wait()
        pltpu.make_async_copy(
            eo_hbm.at[i1_v.at[pl.ds(ci * _CHT, _CHT)]], b_v, sem).wait()

    def compute_store(ci, a_v, b_v, y_v):
        w0c = w0_v[pl.ds(ci * _CHT, L)]
        w1c = w1_v[pl.ds(ci * _CHT, L)]
        for t in range(_CHT):
            w0s = _lane_broadcast(w0c, t)
            w1s = _lane_broadcast(w1c, t)

            def feat(j, carry2, _t=t, _w0=w0s, _w1=w1s):
                a = a_v[_t, pl.ds(j * L, L)]
                b = b_v[_t, pl.ds(j * L, L)]
                y_v[_t, pl.ds(j * L, L)] = _w0 * a + _w1 * b
                return carry2
            lax.fori_loop(0, D // L, feat, 0)
        pltpu.sync_copy(y_v, y_hbm.at[pl.ds(base + ci * _CHT, _CHT)])

    gather(0, a0_v, b0_v, sem0)

    @pl.loop(0, _NCHT, step=2)
    def _(ci):
        gather(ci + 1, a1_v, b1_v, sem1)
        drain(ci, a0_v, b0_v, sem0)
        compute_store(ci, a0_v, b0_v, y0_v)

        @pl.when(ci + 2 < _NCHT)
        def _():
            gather(ci + 2, a0_v, b0_v, sem0)
        drain(ci + 1, a1_v, b1_v, sem1)
        compute_store(ci + 1, a1_v, b1_v, y1_v)


def _combine(eo, s0c, s1c, w0, w1):
    return pl.kernel(
        _combine_body,
        out_type=jax.ShapeDtypeStruct((T, D), jnp.float32),
        scratch_types=[pltpu.VMEM((_TOK_W,), jnp.int32),
                       pltpu.VMEM((_TOK_W,), jnp.int32),
                       pltpu.VMEM((_TOK_W,), jnp.float32),
                       pltpu.VMEM((_TOK_W,), jnp.float32),
                       pltpu.VMEM((_CHT, D), jnp.float32),
                       pltpu.VMEM((_CHT, D), jnp.float32),
                       pltpu.VMEM((_CHT, D), jnp.float32),
                       pltpu.VMEM((_CHT, D), jnp.float32),
                       pltpu.VMEM((_CHT, D), jnp.float32),
                       pltpu.VMEM((_CHT, D), jnp.float32),
                       pltpu.SemaphoreType.DMA,
                       pltpu.SemaphoreType.DMA],
        **_SC_MESH,
    )(eo, s0c, s1c, w0, w1)


def kernel(x, Wr, W1, W2):
    wr_pad = jnp.pad(Wr, ((0, 0), (0, 128 - E)))
    s0r, s1r, s0c, s1c, w0, w1 = _router(x, wr_pad)
    tfs = _invert(s0r.reshape(T), s1r.reshape(T))
    buf = _dispatch(x, tfs)
    eo = _ffn(buf.reshape(E, C, D), W1.astype(jnp.bfloat16),
              W2.astype(jnp.bfloat16))
    return _combine(eo.reshape(S, D), s0c.reshape(T), s1c.reshape(T),
                    w0.reshape(T), w1.reshape(T))
